# baseline (speedup 1.0000x reference)
import functools

import jax
import jax.numpy as jnp
from jax import lax
from jax.experimental import pallas as pl
from jax.experimental.pallas import tpu as pltpu
from jax.experimental.pallas import tpu_sc as plsc

N = 10000
E = 320000
D_IN = 128
D_HID = 64
D_OUT = 128

NC, NS = 2, 16
NW = NC * NS
BLK = 128
EP = 2560 * BLK
BPT = (EP // BLK) // NW
NR = 10240
RPT = NR // NS
NSLOT = 2
HSLOT = 4


def _sc_hist_body(col_hbm, zero_hbm, out_hbm, call, ones, acc_sh, *sems):
    c = lax.axis_index("c")
    s = lax.axis_index("s")
    w = c * NS + s
    for i in range(BLK // 16):
        ones[pl.ds(i * 16, 16)] = jnp.ones((16,), jnp.float32)
    pltpu.sync_copy(zero_hbm.at[pl.ds(s * RPT, RPT)],
                    acc_sh.at[pl.ds(s * RPT, RPT)])
    pltpu.sync_copy(col_hbm.at[pl.ds(w * BPT, BPT)], call)
    plsc.subcore_barrier()

    def body(m, carry):
        for k in range(HSLOT):
            b = m * HSLOT + k

            @pl.when(m > 0)
            def _wait():
                pltpu.make_async_copy(ones, acc_sh.at[call.at[0]],
                                      sems[k]).wait()

            pltpu.async_copy(ones, acc_sh.at[call.at[b]], sems[k], add=True)
        return carry

    lax.fori_loop(0, BPT // HSLOT, body, 0)
    for k in range(HSLOT):
        pltpu.make_async_copy(ones, acc_sh.at[call.at[0]], sems[k]).wait()
    plsc.subcore_barrier()
    pltpu.sync_copy(acc_sh.at[pl.ds(s * RPT, RPT)],
                    out_hbm.at[c, pl.ds(s * RPT, RPT)])


def _sc_prop_body(y_hbm, row_hbm, col_hbm, zero_hbm, out_hbm,
                  rall, call, acc_sh, tab_sh, *bufs_sems):
    gbufs = bufs_sems[:NSLOT]
    semg = bufs_sems[NSLOT:2 * NSLOT]
    sems = bufs_sems[2 * NSLOT:]
    c = lax.axis_index("c")
    s = lax.axis_index("s")
    w = c * NS + s
    pltpu.sync_copy(zero_hbm.at[pl.ds(s * RPT, RPT)],
                    acc_sh.at[pl.ds(s * RPT, RPT)])
    pltpu.sync_copy(y_hbm.at[pl.ds(s * RPT, RPT)],
                    tab_sh.at[pl.ds(s * RPT, RPT)])
    pltpu.sync_copy(row_hbm.at[pl.ds(w * BPT, BPT)], rall)
    pltpu.sync_copy(col_hbm.at[pl.ds(w * BPT, BPT)], call)
    plsc.subcore_barrier()

    def body(m, carry):
        gd = []
        for k in range(NSLOT):
            b = m * NSLOT + k

            @pl.when(m > 0)
            def _wait():
                pltpu.make_async_copy(gbufs[k], acc_sh.at[call.at[0]],
                                      sems[k]).wait()

            gd.append(pltpu.async_copy(tab_sh.at[rall.at[b]], gbufs[k],
                                       semg[k]))
        for k in range(NSLOT):
            b = m * NSLOT + k
            gd[k].wait()
            pltpu.async_copy(gbufs[k], acc_sh.at[call.at[b]], sems[k],
                             add=True)
        return carry

    lax.fori_loop(0, BPT // NSLOT, body, 0)
    for k in range(NSLOT):
        pltpu.make_async_copy(gbufs[k], acc_sh.at[call.at[0]], sems[k]).wait()
    plsc.subcore_barrier()
    pltpu.sync_copy(acc_sh.at[pl.ds(s * RPT, RPT)],
                    out_hbm.at[c, pl.ds(s * RPT, RPT)])


@functools.cache
def _sc_kernels():
    mesh = plsc.VectorSubcoreMesh(core_axis_name="c", subcore_axis_name="s",
                                  num_cores=NC, num_subcores=NS)
    params = pltpu.CompilerParams(use_tc_tiling_on_sc=False)
    hist = pl.kernel(
        _sc_hist_body,
        out_type=jax.ShapeDtypeStruct((NC, NR), jnp.float32),
        mesh=mesh,
        scratch_types=[
            pltpu.VMEM((BPT, BLK), jnp.int32),
            pltpu.VMEM((BLK,), jnp.float32),
            pltpu.VMEM_SHARED((NR,), jnp.float32),
        ] + [pltpu.SemaphoreType.DMA] * HSLOT,
        compiler_params=params,
    )
    prop = pl.kernel(
        _sc_prop_body,
        out_type=jax.ShapeDtypeStruct((NC, NR, D_HID), jnp.float32),
        mesh=mesh,
        scratch_types=[
            pltpu.VMEM((BPT, BLK), jnp.int32),
            pltpu.VMEM((BPT, BLK), jnp.int32),
            pltpu.VMEM_SHARED((NR, D_HID), jnp.float32),
            pltpu.VMEM_SHARED((NR, D_HID), jnp.float32),
        ] + [pltpu.VMEM((BLK, D_HID), jnp.float32)] * NSLOT
          + [pltpu.SemaphoreType.DMA] * (2 * NSLOT),
        compiler_params=params,
    )
    return hist, prop


def _sc_hist(col2d, zero_pad):
    return _sc_kernels()[0](col2d, zero_pad)


def _sc_prop(y, row2d, col2d, zero_nd):
    return _sc_kernels()[1](y, row2d, col2d, zero_nd)


RB = 2048
_HI = jax.lax.Precision.HIGHEST


def _tc1_body(x_ref, w_ref, dp0_ref, dp1_ref, y_ref, dis_ref):
    deg = dp0_ref[...] + dp1_ref[...] + 1.0
    dis = lax.rsqrt(deg)
    dis_ref[...] = dis
    y = jnp.dot(x_ref[...], w_ref[...], precision=_HI,
                preferred_element_type=jnp.float32)
    y_ref[...] = y * dis


def _tc1(x, w1t, dp0, dp1):
    return pl.pallas_call(
        _tc1_body,
        grid=(NR // RB,),
        in_specs=[
            pl.BlockSpec((RB, D_IN), lambda i: (i, 0)),
            pl.BlockSpec((D_IN, D_HID), lambda i: (0, 0)),
            pl.BlockSpec((RB, 1), lambda i: (i, 0)),
            pl.BlockSpec((RB, 1), lambda i: (i, 0)),
        ],
        out_specs=[
            pl.BlockSpec((RB, D_HID), lambda i: (i, 0)),
            pl.BlockSpec((RB, 1), lambda i: (i, 0)),
        ],
        out_shape=[
            jax.ShapeDtypeStruct((NR, D_HID), jnp.float32),
            jax.ShapeDtypeStruct((NR, 1), jnp.float32),
        ],
    )(x, w1t, dp0, dp1)


def _tc2_body(a_ref, y_ref, dis_ref, b1_ref, h_ref):
    tot = a_ref[0] + a_ref[1] + y_ref[...]
    p = tot * dis_ref[...] + b1_ref[...]
    h = jnp.maximum(p, 0.0)
    h_ref[...] = h * dis_ref[...]


def _tc2(a1, y1t, dis, b1r):
    return pl.pallas_call(
        _tc2_body,
        grid=(NR // RB,),
        in_specs=[
            pl.BlockSpec((NC, RB, D_HID), lambda i: (0, i, 0)),
            pl.BlockSpec((RB, D_HID), lambda i: (i, 0)),
            pl.BlockSpec((RB, 1), lambda i: (i, 0)),
            pl.BlockSpec((1, D_HID), lambda i: (0, 0)),
        ],
        out_specs=pl.BlockSpec((RB, D_HID), lambda i: (i, 0)),
        out_shape=jax.ShapeDtypeStruct((NR, D_HID), jnp.float32),
    )(a1, y1t, dis, b1r)


def _tc3_body(a_ref, h_ref, dis_ref, w_ref, b2_ref, o_ref):
    p = (a_ref[0] + a_ref[1] + h_ref[...]) * dis_ref[...]
    z = jnp.dot(p, w_ref[...], precision=_HI,
                preferred_element_type=jnp.float32) + b2_ref[...]
    m = jnp.max(z, axis=1, keepdims=True)
    e = jnp.exp(z - m)
    ssum = jnp.sum(e, axis=1, keepdims=True)
    o_ref[...] = (z - m) - jnp.log(ssum)


def _tc3(a2, ht, dis, w2t, b2r):
    return pl.pallas_call(
        _tc3_body,
        grid=(NR // RB,),
        in_specs=[
            pl.BlockSpec((NC, RB, D_HID), lambda i: (0, i, 0)),
            pl.BlockSpec((RB, D_HID), lambda i: (i, 0)),
            pl.BlockSpec((RB, 1), lambda i: (i, 0)),
            pl.BlockSpec((D_HID, D_OUT), lambda i: (0, 0)),
            pl.BlockSpec((1, D_OUT), lambda i: (0, 0)),
        ],
        out_specs=pl.BlockSpec((RB, D_OUT), lambda i: (i, 0)),
        out_shape=jax.ShapeDtypeStruct((N, D_OUT), jnp.float32),
    )(a2, ht, dis, w2t, b2r)


def kernel(x, edge_index, W1, b1, W2, b2):
    ar = jnp.arange(EP - E, dtype=jnp.int32)
    row2d = jnp.concatenate(
        [edge_index[0], ar % N]).reshape(-1, BLK)
    col2d = jnp.concatenate(
        [edge_index[1], N + ar % (NR - N)]).reshape(-1, BLK)
    zero_pad = jnp.zeros((NR,), jnp.float32)
    zero_nd = jnp.zeros((NR, D_HID), jnp.float32)

    dp = _sc_hist(col2d, zero_pad)
    dp0 = dp[0, :, None]
    dp1 = dp[1, :, None]

    y1t, dis = _tc1(x, W1.T, dp0, dp1)
    a1 = _sc_prop(y1t, row2d, col2d, zero_nd)
    ht = _tc2(a1, y1t, dis, b1[None, :])
    a2 = _sc_prop(ht, row2d, col2d, zero_nd)
    return _tc3(a2, ht, dis, W2.T, b2[None, :])


# speedup vs baseline: 1.2388x; 1.2388x over previous
import functools

import jax
import jax.numpy as jnp
from jax import lax
from jax.experimental import pallas as pl
from jax.experimental.pallas import tpu as pltpu
from jax.experimental.pallas import tpu_sc as plsc

N = 10000
E = 320000
D_IN = 128
D_HID = 64
D_OUT = 128

NC, NS = 2, 16
NW = NC * NS
BLK = 128
EP = 2560 * BLK
BPT = (EP // BLK) // NW
NR = 10240
RPT = NR // NS
NSLOT = 8
HSLOT = 4


def _sc_hist_body(col_hbm, out_hbm, call, ones, zbuf, acc_sh, *sems):
    c = lax.axis_index("c")
    s = lax.axis_index("s")
    w = c * NS + s
    for i in range(BLK // 16):
        ones[pl.ds(i * 16, 16)] = jnp.ones((16,), jnp.float32)

    def zero(i, carry):
        zbuf[pl.ds(i * 16, 16)] = jnp.zeros((16,), jnp.float32)
        return carry

    lax.fori_loop(0, RPT // 16, zero, 0)
    pltpu.sync_copy(zbuf, acc_sh.at[pl.ds(s * RPT, RPT)])
    pltpu.sync_copy(col_hbm.at[pl.ds(w * BPT, BPT)], call)
    plsc.subcore_barrier()

    def body(m, carry):
        for k in range(HSLOT):
            b = m * HSLOT + k

            @pl.when(m > 0)
            def _wait():
                pltpu.make_async_copy(ones, acc_sh.at[call.at[0]],
                                      sems[k]).wait()

            pltpu.async_copy(ones, acc_sh.at[call.at[b]], sems[k], add=True)
        return carry

    lax.fori_loop(0, BPT // HSLOT, body, 0)
    for k in range(HSLOT):
        pltpu.make_async_copy(ones, acc_sh.at[call.at[0]], sems[k]).wait()
    plsc.subcore_barrier()
    pltpu.sync_copy(acc_sh.at[pl.ds(s * RPT, RPT)],
                    out_hbm.at[c, pl.ds(s * RPT, RPT)])


def _sc_prop_body(y_hbm, row_hbm, col_hbm, out_hbm,
                  rall, call, acc_sh, *bufs_sems):
    gbufs = bufs_sems[:NSLOT]
    semg = bufs_sems[NSLOT:2 * NSLOT]
    sems = bufs_sems[2 * NSLOT:]
    c = lax.axis_index("c")
    s = lax.axis_index("s")
    w = c * NS + s

    def zero(r, carry):
        for j in range(D_HID // 16):
            gbufs[0][r, pl.ds(j * 16, 16)] = jnp.zeros((16,), jnp.float32)
        return carry

    lax.fori_loop(0, BLK, zero, 0)
    for j in range(RPT // BLK):
        pltpu.sync_copy(gbufs[0],
                        acc_sh.at[pl.ds(s * RPT + j * BLK, BLK)])
    pltpu.sync_copy(row_hbm.at[pl.ds(w * BPT, BPT)], rall)
    pltpu.sync_copy(col_hbm.at[pl.ds(w * BPT, BPT)], call)
    plsc.subcore_barrier()

    def body(m, carry):
        gd = []
        for k in range(NSLOT):
            b = m * NSLOT + k

            @pl.when(m > 0)
            def _wait():
                pltpu.make_async_copy(gbufs[k], acc_sh.at[call.at[0]],
                                      sems[k]).wait()

            gd.append(pltpu.async_copy(y_hbm.at[rall.at[b]], gbufs[k],
                                       semg[k]))
        for k in range(NSLOT):
            b = m * NSLOT + k
            gd[k].wait()
            pltpu.async_copy(gbufs[k], acc_sh.at[call.at[b]], sems[k],
                             add=True)
        return carry

    lax.fori_loop(0, BPT // NSLOT, body, 0)
    for k in range(NSLOT):
        pltpu.make_async_copy(gbufs[k], acc_sh.at[call.at[0]], sems[k]).wait()
    plsc.subcore_barrier()
    pltpu.sync_copy(acc_sh.at[pl.ds(s * RPT, RPT)],
                    out_hbm.at[c, pl.ds(s * RPT, RPT)])


@functools.cache
def _sc_kernels():
    mesh = plsc.VectorSubcoreMesh(core_axis_name="c", subcore_axis_name="s",
                                  num_cores=NC, num_subcores=NS)
    params = pltpu.CompilerParams(use_tc_tiling_on_sc=False)
    hist = pl.kernel(
        _sc_hist_body,
        out_type=jax.ShapeDtypeStruct((NC, NR), jnp.float32),
        mesh=mesh,
        scratch_types=[
            pltpu.VMEM((BPT, BLK), jnp.int32),
            pltpu.VMEM((BLK,), jnp.float32),
            pltpu.VMEM((RPT,), jnp.float32),
            pltpu.VMEM_SHARED((NR,), jnp.float32),
        ] + [pltpu.SemaphoreType.DMA] * HSLOT,
        compiler_params=params,
    )
    prop = pl.kernel(
        _sc_prop_body,
        out_type=jax.ShapeDtypeStruct((NC, NR, D_HID), jnp.float32),
        mesh=mesh,
        scratch_types=[
            pltpu.VMEM((BPT, BLK), jnp.int32),
            pltpu.VMEM((BPT, BLK), jnp.int32),
            pltpu.VMEM_SHARED((NR, D_HID), jnp.float32),
        ] + [pltpu.VMEM((BLK, D_HID), jnp.float32)] * NSLOT
          + [pltpu.SemaphoreType.DMA] * (2 * NSLOT),
        compiler_params=params,
    )
    return hist, prop


def _sc_hist(col2d):
    return _sc_kernels()[0](col2d)


def _sc_prop(y, row2d, col2d):
    return _sc_kernels()[1](y, row2d, col2d)


RB = 2048
_HI = jax.lax.Precision.HIGHEST


def _tc1_body(x_ref, w_ref, dp0_ref, dp1_ref, y_ref, dis_ref):
    deg = dp0_ref[...] + dp1_ref[...] + 1.0
    dis = lax.rsqrt(deg)
    dis_ref[...] = dis
    y = lax.dot_general(x_ref[...], w_ref[...], (((1,), (1,)), ((), ())),
                        precision=_HI, preferred_element_type=jnp.float32)
    y_ref[...] = y * dis


def _tc1(x, w1, dp0, dp1):
    return pl.pallas_call(
        _tc1_body,
        grid=(NR // RB,),
        in_specs=[
            pl.BlockSpec((RB, D_IN), lambda i: (i, 0)),
            pl.BlockSpec((D_HID, D_IN), lambda i: (0, 0)),
            pl.BlockSpec((RB, 1), lambda i: (i, 0)),
            pl.BlockSpec((RB, 1), lambda i: (i, 0)),
        ],
        out_specs=[
            pl.BlockSpec((RB, D_HID), lambda i: (i, 0)),
            pl.BlockSpec((RB, 1), lambda i: (i, 0)),
        ],
        out_shape=[
            jax.ShapeDtypeStruct((NR, D_HID), jnp.float32),
            jax.ShapeDtypeStruct((NR, 1), jnp.float32),
        ],
    )(x, w1, dp0, dp1)


def _tc2_body(a_ref, y_ref, dis_ref, b1_ref, h_ref):
    tot = a_ref[0] + a_ref[1] + y_ref[...]
    p = tot * dis_ref[...] + b1_ref[...]
    h = jnp.maximum(p, 0.0)
    h_ref[...] = h * dis_ref[...]


def _tc2(a1, y1t, dis, b1r):
    return pl.pallas_call(
        _tc2_body,
        grid=(NR // RB,),
        in_specs=[
            pl.BlockSpec((NC, RB, D_HID), lambda i: (0, i, 0)),
            pl.BlockSpec((RB, D_HID), lambda i: (i, 0)),
            pl.BlockSpec((RB, 1), lambda i: (i, 0)),
            pl.BlockSpec((1, D_HID), lambda i: (0, 0)),
        ],
        out_specs=pl.BlockSpec((RB, D_HID), lambda i: (i, 0)),
        out_shape=jax.ShapeDtypeStruct((NR, D_HID), jnp.float32),
    )(a1, y1t, dis, b1r)


def _tc3_body(a_ref, h_ref, dis_ref, w_ref, b2_ref, o_ref):
    p = (a_ref[0] + a_ref[1] + h_ref[...]) * dis_ref[...]
    z = lax.dot_general(p, w_ref[...], (((1,), (1,)), ((), ())),
                        precision=_HI,
                        preferred_element_type=jnp.float32) + b2_ref[...]
    m = jnp.max(z, axis=1, keepdims=True)
    e = jnp.exp(z - m)
    ssum = jnp.sum(e, axis=1, keepdims=True)
    o_ref[...] = (z - m) - jnp.log(ssum)


def _tc3(a2, ht, dis, w2, b2r):
    return pl.pallas_call(
        _tc3_body,
        grid=(NR // RB,),
        in_specs=[
            pl.BlockSpec((NC, RB, D_HID), lambda i: (0, i, 0)),
            pl.BlockSpec((RB, D_HID), lambda i: (i, 0)),
            pl.BlockSpec((RB, 1), lambda i: (i, 0)),
            pl.BlockSpec((D_OUT, D_HID), lambda i: (0, 0)),
            pl.BlockSpec((1, D_OUT), lambda i: (0, 0)),
        ],
        out_specs=pl.BlockSpec((RB, D_OUT), lambda i: (i, 0)),
        out_shape=jax.ShapeDtypeStruct((N, D_OUT), jnp.float32),
    )(a2, ht, dis, w2, b2r)


def kernel(x, edge_index, W1, b1, W2, b2):
    ar = jnp.arange(EP - E, dtype=jnp.int32)
    row2d = jnp.concatenate(
        [edge_index[0], ar % N]).reshape(-1, BLK)
    col2d = jnp.concatenate(
        [edge_index[1], N + ar % (NR - N)]).reshape(-1, BLK)
    dp = _sc_hist(col2d)
    dp0 = dp[0, :, None]
    dp1 = dp[1, :, None]

    y1t, dis = _tc1(x, W1, dp0, dp1)
    a1 = _sc_prop(y1t, row2d, col2d)
    ht = _tc2(a1, y1t, dis, b1[None, :])
    a2 = _sc_prop(ht, row2d, col2d)
    return _tc3(a2, ht, dis, W2, b2[None, :])


# speedup vs baseline: 1.2445x; 1.0047x over previous
import functools

import jax
import jax.numpy as jnp
from jax import lax
from jax.experimental import pallas as pl
from jax.experimental.pallas import tpu as pltpu
from jax.experimental.pallas import tpu_sc as plsc

N = 10000
E = 320000
D_IN = 128
D_HID = 64
D_OUT = 128

NC, NS = 2, 16
NW = NC * NS
BLK = 128
EP = 2560 * BLK
BPT = (EP // BLK) // NW
NR = 10240
RPT = NR // NS
NSLOT = 8
HSLOT = 4


def _sc_hist_body(col_hbm, out_hbm, call, ones, zbuf, acc_sh, *sems):
    c = lax.axis_index("c")
    s = lax.axis_index("s")
    w = c * NS + s
    for i in range(BLK // 16):
        ones[pl.ds(i * 16, 16)] = jnp.ones((16,), jnp.float32)

    def zero(i, carry):
        zbuf[pl.ds(i * 16, 16)] = jnp.zeros((16,), jnp.float32)
        return carry

    lax.fori_loop(0, RPT // 16, zero, 0)
    pltpu.sync_copy(zbuf, acc_sh.at[pl.ds(s * RPT, RPT)])
    pltpu.sync_copy(col_hbm.at[pl.ds(w * BPT, BPT)], call)
    plsc.subcore_barrier()

    def body(m, carry):
        for k in range(HSLOT):
            b = m * HSLOT + k

            @pl.when(m > 0)
            def _wait():
                pltpu.make_async_copy(ones, acc_sh.at[call.at[0]],
                                      sems[k]).wait()

            pltpu.async_copy(ones, acc_sh.at[call.at[b]], sems[k], add=True)
        return carry

    lax.fori_loop(0, BPT // HSLOT, body, 0)
    for k in range(HSLOT):
        pltpu.make_async_copy(ones, acc_sh.at[call.at[0]], sems[k]).wait()
    plsc.subcore_barrier()
    pltpu.sync_copy(acc_sh.at[pl.ds(s * RPT, RPT)],
                    out_hbm.at[c, pl.ds(s * RPT, RPT)])


def _sc_prop_body(y_hbm, row_hbm, col_hbm, out_hbm,
                  rall, call, acc_sh, *bufs_sems):
    gbufs = bufs_sems[:NSLOT]
    semg = bufs_sems[NSLOT:2 * NSLOT]
    sems = bufs_sems[2 * NSLOT:]
    c = lax.axis_index("c")
    s = lax.axis_index("s")
    w = c * NS + s

    @pl.when(c == 0)
    def _init_self():
        pltpu.sync_copy(y_hbm.at[pl.ds(s * RPT, RPT)],
                        acc_sh.at[pl.ds(s * RPT, RPT)])

    @pl.when(c != 0)
    def _init_zero():
        def zero(r, carry):
            for j in range(D_HID // 16):
                gbufs[0][r, pl.ds(j * 16, 16)] = jnp.zeros((16,), jnp.float32)
            return carry

        lax.fori_loop(0, BLK, zero, 0)
        for j in range(RPT // BLK):
            pltpu.sync_copy(gbufs[0],
                            acc_sh.at[pl.ds(s * RPT + j * BLK, BLK)])
    pltpu.sync_copy(row_hbm.at[pl.ds(w * BPT, BPT)], rall)
    pltpu.sync_copy(col_hbm.at[pl.ds(w * BPT, BPT)], call)
    plsc.subcore_barrier()

    def body(m, carry):
        gd = []
        for k in range(NSLOT):
            b = m * NSLOT + k

            @pl.when(m > 0)
            def _wait():
                pltpu.make_async_copy(gbufs[k], acc_sh.at[call.at[0]],
                                      sems[k]).wait()

            gd.append(pltpu.async_copy(y_hbm.at[rall.at[b]], gbufs[k],
                                       semg[k]))
        for k in range(NSLOT):
            b = m * NSLOT + k
            gd[k].wait()
            pltpu.async_copy(gbufs[k], acc_sh.at[call.at[b]], sems[k],
                             add=True)
        return carry

    lax.fori_loop(0, BPT // NSLOT, body, 0)
    for k in range(NSLOT):
        pltpu.make_async_copy(gbufs[k], acc_sh.at[call.at[0]], sems[k]).wait()
    plsc.subcore_barrier()
    pltpu.sync_copy(acc_sh.at[pl.ds(s * RPT, RPT)],
                    out_hbm.at[c, pl.ds(s * RPT, RPT)])


@functools.cache
def _sc_kernels():
    mesh = plsc.VectorSubcoreMesh(core_axis_name="c", subcore_axis_name="s",
                                  num_cores=NC, num_subcores=NS)
    params = pltpu.CompilerParams(use_tc_tiling_on_sc=False)
    hist = pl.kernel(
        _sc_hist_body,
        out_type=jax.ShapeDtypeStruct((NC, NR), jnp.float32),
        mesh=mesh,
        scratch_types=[
            pltpu.VMEM((BPT, BLK), jnp.int32),
            pltpu.VMEM((BLK,), jnp.float32),
            pltpu.VMEM((RPT,), jnp.float32),
            pltpu.VMEM_SHARED((NR,), jnp.float32),
        ] + [pltpu.SemaphoreType.DMA] * HSLOT,
        compiler_params=params,
    )
    prop = pl.kernel(
        _sc_prop_body,
        out_type=jax.ShapeDtypeStruct((NC, NR, D_HID), jnp.float32),
        mesh=mesh,
        scratch_types=[
            pltpu.VMEM((BPT, BLK), jnp.int32),
            pltpu.VMEM((BPT, BLK), jnp.int32),
            pltpu.VMEM_SHARED((NR, D_HID), jnp.float32),
        ] + [pltpu.VMEM((BLK, D_HID), jnp.float32)] * NSLOT
          + [pltpu.SemaphoreType.DMA] * (2 * NSLOT),
        compiler_params=params,
    )
    return hist, prop


def _sc_hist(col2d):
    return _sc_kernels()[0](col2d)


def _sc_prop(y, row2d, col2d):
    return _sc_kernels()[1](y, row2d, col2d)


RB = 2048
_HI = jax.lax.Precision.HIGHEST


def _tc1_body(x_ref, w_ref, dp0_ref, dp1_ref, y_ref, dis_ref):
    deg = dp0_ref[...] + dp1_ref[...] + 1.0
    dis = lax.rsqrt(deg)
    dis_ref[...] = dis
    y = lax.dot_general(x_ref[...], w_ref[...], (((1,), (1,)), ((), ())),
                        precision=_HI, preferred_element_type=jnp.float32)
    y_ref[...] = y * dis


def _tc1(x, w1, dp0, dp1):
    return pl.pallas_call(
        _tc1_body,
        grid=(NR // RB,),
        in_specs=[
            pl.BlockSpec((RB, D_IN), lambda i: (i, 0)),
            pl.BlockSpec((D_HID, D_IN), lambda i: (0, 0)),
            pl.BlockSpec((RB, 1), lambda i: (i, 0)),
            pl.BlockSpec((RB, 1), lambda i: (i, 0)),
        ],
        out_specs=[
            pl.BlockSpec((RB, D_HID), lambda i: (i, 0)),
            pl.BlockSpec((RB, 1), lambda i: (i, 0)),
        ],
        out_shape=[
            jax.ShapeDtypeStruct((NR, D_HID), jnp.float32),
            jax.ShapeDtypeStruct((NR, 1), jnp.float32),
        ],
    )(x, w1, dp0, dp1)


def _tc2_body(a_ref, dis_ref, b1_ref, h_ref):
    tot = a_ref[0] + a_ref[1]
    p = tot * dis_ref[...] + b1_ref[...]
    h = jnp.maximum(p, 0.0)
    h_ref[...] = h * dis_ref[...]


def _tc2(a1, dis, b1r):
    return pl.pallas_call(
        _tc2_body,
        grid=(NR // RB,),
        in_specs=[
            pl.BlockSpec((NC, RB, D_HID), lambda i: (0, i, 0)),
            pl.BlockSpec((RB, 1), lambda i: (i, 0)),
            pl.BlockSpec((1, D_HID), lambda i: (0, 0)),
        ],
        out_specs=pl.BlockSpec((RB, D_HID), lambda i: (i, 0)),
        out_shape=jax.ShapeDtypeStruct((NR, D_HID), jnp.float32),
    )(a1, dis, b1r)


def _tc3_body(a_ref, dis_ref, w_ref, b2_ref, o_ref):
    p = (a_ref[0] + a_ref[1]) * dis_ref[...]
    z = lax.dot_general(p, w_ref[...], (((1,), (1,)), ((), ())),
                        precision=_HI,
                        preferred_element_type=jnp.float32) + b2_ref[...]
    m = jnp.max(z, axis=1, keepdims=True)
    e = jnp.exp(z - m)
    ssum = jnp.sum(e, axis=1, keepdims=True)
    o_ref[...] = (z - m) - jnp.log(ssum)


def _tc3(a2, dis, w2, b2r):
    return pl.pallas_call(
        _tc3_body,
        grid=(NR // RB,),
        in_specs=[
            pl.BlockSpec((NC, RB, D_HID), lambda i: (0, i, 0)),
            pl.BlockSpec((RB, 1), lambda i: (i, 0)),
            pl.BlockSpec((D_OUT, D_HID), lambda i: (0, 0)),
            pl.BlockSpec((1, D_OUT), lambda i: (0, 0)),
        ],
        out_specs=pl.BlockSpec((RB, D_OUT), lambda i: (i, 0)),
        out_shape=jax.ShapeDtypeStruct((N, D_OUT), jnp.float32),
    )(a2, dis, w2, b2r)


def kernel(x, edge_index, W1, b1, W2, b2):
    ar = jnp.arange(EP - E, dtype=jnp.int32)
    row2d = jnp.concatenate(
        [edge_index[0], ar % N]).reshape(-1, BLK)
    col2d = jnp.concatenate(
        [edge_index[1], N + ar % (NR - N)]).reshape(-1, BLK)
    dp = _sc_hist(col2d)
    dp0 = dp[0, :, None]
    dp1 = dp[1, :, None]

    y1t, dis = _tc1(x, W1, dp0, dp1)
    a1 = _sc_prop(y1t, row2d, col2d)
    ht = _tc2(a1, dis, b1[None, :])
    a2 = _sc_prop(ht, row2d, col2d)
    return _tc3(a2, dis, W2, b2[None, :])
